# baseline (speedup 1.0000x reference)
import functools

import jax
import jax.numpy as jnp
from jax import lax
from jax.experimental import pallas as pl
from jax.experimental.pallas import tpu as pltpu
from jax.experimental.pallas import tpu_sc as plsc

NC = 2
NS = 16
NW = NC * NS
L = 16


@functools.lru_cache(maxsize=None)
def _make_kernel(B, S, V, D, C):
    s_per_w = S // NW
    chunks = s_per_w // C
    ncol = D // L

    mesh = plsc.VectorSubcoreMesh(core_axis_name="c", subcore_axis_name="s")

    @functools.partial(
        pl.kernel,
        mesh=mesh,
        out_type=jax.ShapeDtypeStruct((B * S, D), jnp.float32),
        scratch_types=[
            pltpu.VMEM((B, s_per_w), jnp.int32),
            pltpu.VMEM((2 * B, C, D), jnp.float32),
            pltpu.VMEM((2, C, D), jnp.float32),
            pltpu.SemaphoreType.DMA((2 * B,)),
            pltpu.SemaphoreType.DMA((2 * B,)),
            pltpu.SemaphoreType.DMA((2,)),
        ],
    )
    def emb_kernel(ids_hbm, tok_hbm, pos_hbm, out_hbm, idx_v, tokb, posb, gsem, ssem, psem):
        wid = lax.axis_index("s") * NC + lax.axis_index("c")
        s0 = wid * s_per_w

        for b in range(B):
            pltpu.sync_copy(ids_hbm.at[pl.ds(b * S + s0, s_per_w)], idx_v.at[b])

        def gather(k, g, b):
            pltpu.async_copy(
                tok_hbm.at[idx_v.at[b, pl.ds(k * C, C)]],
                tokb.at[g * B + b],
                gsem.at[g * B + b],
            )

        def gather_wait(g, b):
            pltpu.make_async_copy(
                tok_hbm.at[pl.ds(0, C)], tokb.at[g * B + b], gsem.at[g * B + b]
            ).wait()

        def scatter(k, g, b):
            pltpu.async_copy(
                tokb.at[g * B + b],
                out_hbm.at[pl.ds(b * S + s0 + k * C, C)],
                ssem.at[g * B + b],
            )

        def scatter_wait(g, b):
            pltpu.make_async_copy(
                tokb.at[g * B + b], out_hbm.at[pl.ds(b * S + s0, C)], ssem.at[g * B + b]
            ).wait()

        def pos_load(k, pb):
            pltpu.async_copy(pos_hbm.at[pl.ds(s0 + k * C, C)], posb.at[pb], psem.at[pb])

        def pos_wait(pb):
            pltpu.make_async_copy(
                pos_hbm.at[pl.ds(s0, C)], posb.at[pb], psem.at[pb]
            ).wait()

        pos_load(0, 0)
        for b in range(B):
            gather(0, 0, b)

        def outer(i, carry):
            for kk in range(2):
                k = i * 2 + kk
                g = kk
                pb = kk
                pos_wait(pb)
                if kk == 0:
                    pos_load(k + 1, 1 - pb)
                else:
                    @pl.when(i < chunks // 2 - 1)
                    def _():
                        pos_load(k + 1, 1 - pb)

                if kk == 0:
                    @pl.when(i >= 1)
                    def _():
                        for b in range(B):
                            scatter_wait(1 - g, b)
                    for b in range(B):
                        gather(k + 1, 1 - g, b)
                else:
                    @pl.when(i < chunks // 2 - 1)
                    def _():
                        for b in range(B):
                            scatter_wait(1 - g, b)
                            gather(k + 1, 1 - g, b)

                for b in range(B):
                    gather_wait(g, b)

                def row_body(r, c2):
                    for c in range(ncol):
                        sl = pl.ds(c * L, L)
                        pv = posb[pb, r, sl]
                        for b in range(B):
                            tokb[g * B + b, r, sl] = tokb[g * B + b, r, sl] + pv
                    return c2

                lax.fori_loop(0, C, row_body, 0)

                for b in range(B):
                    scatter(k, g, b)
            return carry

        lax.fori_loop(0, chunks // 2, outer, 0)

        for g in range(2):
            for b in range(B):
                scatter_wait(g, b)

    return emb_kernel


def kernel(input_ids, token_embeddings, position_embeddings):
    B, S = input_ids.shape
    V, D = token_embeddings.shape
    ids = input_ids.reshape(-1).astype(jnp.int32)
    k = _make_kernel(B, S, V, D, 16)
    out = k(ids, token_embeddings, position_embeddings)
    return out.reshape(B, S, D)


# speedup vs baseline: 1.4375x; 1.4375x over previous
import functools

import jax
import jax.numpy as jnp
from jax import lax
from jax.experimental import pallas as pl
from jax.experimental.pallas import tpu as pltpu
from jax.experimental.pallas import tpu_sc as plsc

NC = 2
NS = 16
NW = NC * NS
L = 16


@functools.lru_cache(maxsize=None)
def _make_kernel(B, S, V, D, C):
    s_per_w = S // NW
    chunks = s_per_w // C
    ncol = D // L

    mesh = plsc.VectorSubcoreMesh(core_axis_name="c", subcore_axis_name="s")

    @functools.partial(
        pl.kernel,
        mesh=mesh,
        out_type=jax.ShapeDtypeStruct((B * S, D), jnp.float32),
        scratch_types=[
            pltpu.VMEM((B, s_per_w), jnp.int32),
            pltpu.VMEM((B, C, D), jnp.float32),
            pltpu.VMEM((2, C, D), jnp.float32),
            pltpu.SemaphoreType.DMA((B,)),
            pltpu.SemaphoreType.DMA((B,)),
            pltpu.SemaphoreType.DMA((2,)),
        ],
    )
    def emb_kernel(ids_hbm, tok_hbm, pos_hbm, out_hbm, idx_v, tokb, posb, gsem, ssem, psem):
        wid = lax.axis_index("s") * NC + lax.axis_index("c")
        s0 = wid * s_per_w

        for b in range(B):
            pltpu.sync_copy(ids_hbm.at[pl.ds(b * S + s0, s_per_w)], idx_v.at[b])

        def gather(k, b):
            pltpu.async_copy(
                tok_hbm.at[idx_v.at[b, pl.ds(k * C, C)]], tokb.at[b], gsem.at[b]
            )

        def gather_wait(b):
            pltpu.make_async_copy(
                tok_hbm.at[pl.ds(0, C)], tokb.at[b], gsem.at[b]
            ).wait()

        def scatter(k, b):
            pltpu.async_copy(
                tokb.at[b], out_hbm.at[pl.ds(b * S + s0 + k * C, C)], ssem.at[b]
            )

        def scatter_wait(b):
            pltpu.make_async_copy(
                tokb.at[b], out_hbm.at[pl.ds(b * S + s0, C)], ssem.at[b]
            ).wait()

        def pos_load(k, pb):
            pltpu.async_copy(pos_hbm.at[pl.ds(s0 + k * C, C)], posb.at[pb], psem.at[pb])

        def pos_wait(pb):
            pltpu.make_async_copy(
                pos_hbm.at[pl.ds(s0, C)], posb.at[pb], psem.at[pb]
            ).wait()

        pos_load(0, 0)
        gather(0, 0)
        gather(0, 1)

        def outer(i, carry):
            for kk in range(2):
                k = i * 2 + kk
                pb = kk
                pos_wait(pb)
                if kk == 0:
                    pos_load(k + 1, 1 - pb)
                else:
                    @pl.when(i < chunks // 2 - 1)
                    def _():
                        pos_load(k + 1, 1 - pb)

                for b in range(B):
                    gather_wait(b)

                    b2 = (b + 2) % B
                    k2 = k + (b + 2) // B
                    if b < 2:
                        if kk == 0:
                            @pl.when(i >= 1)
                            def _():
                                scatter_wait(b2)
                        else:
                            scatter_wait(b2)
                        gather(k2, b2)
                    else:
                        if kk == 0:
                            scatter_wait(b2)
                            gather(k2, b2)
                        else:
                            @pl.when(i < chunks // 2 - 1)
                            def _():
                                scatter_wait(b2)
                                gather(k2, b2)

                    def row_body(r, c2):
                        for c in range(ncol):
                            sl = pl.ds(c * L, L)
                            tokb[b, r, sl] = tokb[b, r, sl] + posb[pb, r, sl]
                        return c2

                    lax.fori_loop(0, C, row_body, 0)
                    scatter(k, b)
            return carry

        lax.fori_loop(0, chunks // 2, outer, 0)

        for b in range(B):
            scatter_wait(b)

    return emb_kernel


def kernel(input_ids, token_embeddings, position_embeddings):
    B, S = input_ids.shape
    V, D = token_embeddings.shape
    ids = input_ids.reshape(-1).astype(jnp.int32)
    k = _make_kernel(B, S, V, D, 16)
    out = k(ids, token_embeddings, position_embeddings)
    return out.reshape(B, S, D)
